# Initial kernel scaffold; baseline (speedup 1.0000x reference)
import jax
import jax.numpy as jnp
from jax.experimental import pallas as pl


def kernel(input, edge_index, edge_attr, W1, b1, W2, b2):
    raise NotImplementedError("write your pallas kernel here")


# speedup vs baseline: 2.4363x; 2.4363x over previous
import functools

import jax
import jax.numpy as jnp
from jax import lax
from jax.experimental import pallas as pl
from jax.experimental.pallas import tpu as pltpu
from jax.experimental.pallas import tpu_sc as plsc

CH = 128
NW = 32


@functools.lru_cache(maxsize=None)
def _make_fnet(E, Epad, DE, H, D):
    BE = 2048
    grid = (Epad // BE,)

    def body(ea, w1, b1, w2, b2, w_out):
        h = jnp.tanh(jnp.dot(ea[...], w1[...], preferred_element_type=jnp.float32)
                     + b1[...])
        w = jnp.dot(h, w2[...], preferred_element_type=jnp.float32) + b2[...]
        i = pl.program_id(0)
        rows = i * BE + lax.broadcasted_iota(jnp.int32, (BE, 1), 0)
        w_out[...] = jnp.where(rows < E, w, 0.0)

    return pl.pallas_call(
        body,
        grid=grid,
        in_specs=[
            pl.BlockSpec((BE, DE), lambda i: (i, 0)),
            pl.BlockSpec((DE, H), lambda i: (0, 0)),
            pl.BlockSpec((1, H), lambda i: (0, 0)),
            pl.BlockSpec((H, D), lambda i: (0, 0)),
            pl.BlockSpec((1, D), lambda i: (0, 0)),
        ],
        out_specs=pl.BlockSpec((BE, D), lambda i: (i, 0)),
        out_shape=jax.ShapeDtypeStruct((Epad, D), jnp.float32),
    )


@functools.lru_cache(maxsize=None)
def _make_softmax(N, D, BN):
    def body(x, o):
        v = x[...]
        m = jnp.max(v, axis=-1, keepdims=True)
        e = jnp.exp(v - m)
        o[...] = e / jnp.sum(e, axis=-1, keepdims=True)

    return pl.pallas_call(
        body,
        grid=(N // BN,),
        in_specs=[pl.BlockSpec((BN, D), lambda i: (i, 0))],
        out_specs=pl.BlockSpec((BN, D), lambda i: (i, 0)),
        out_shape=jax.ShapeDtypeStruct((N, D), jnp.float32),
    )


@functools.lru_cache(maxsize=None)
def _make_update(N, D, BN, do_softmax):
    def body(x, p0, p1, d0, d1, o):
        deg = d0[...] + d1[...]
        degc = jnp.maximum(deg, 1.0)
        q = x[...] - (p0[...] + p1[...]) / degc
        if do_softmax:
            m = jnp.max(q, axis=-1, keepdims=True)
            e = jnp.exp(q - m)
            q = e / jnp.sum(e, axis=-1, keepdims=True)
        o[...] = q

    return pl.pallas_call(
        body,
        grid=(N // BN,),
        in_specs=[
            pl.BlockSpec((BN, D), lambda i: (i, 0)),
            pl.BlockSpec((BN, D), lambda i: (i, 0)),
            pl.BlockSpec((BN, D), lambda i: (i, 0)),
            pl.BlockSpec((BN, 1), lambda i: (i, 0)),
            pl.BlockSpec((BN, 1), lambda i: (i, 0)),
        ],
        out_specs=pl.BlockSpec((BN, D), lambda i: (i, 0)),
        out_shape=jax.ShapeDtypeStruct((N, D), jnp.float32),
    )


@functools.lru_cache(maxsize=None)
def _make_sc_pass(Npad, D, Epad, with_deg):
    EPT = Epad // NW
    CHUNKS = EPT // CH
    RZ = Npad // 16
    mesh = plsc.VectorSubcoreMesh(core_axis_name="c", subcore_axis_name="s")

    outs = [jax.ShapeDtypeStruct((2, Npad, D), jnp.float32)]
    scratch = [
        pltpu.VMEM((CH,), jnp.int32),
        pltpu.VMEM((CH,), jnp.int32),
        pltpu.VMEM((CH, D), jnp.float32),
        pltpu.VMEM((CH, D), jnp.float32),
        pltpu.VMEM_SHARED((Npad, D), jnp.float32),
        pltpu.SemaphoreType.DMA,
    ]
    if with_deg:
        outs.append(jax.ShapeDtypeStruct((2 * Npad,), jnp.float32))
        scratch += [
            pltpu.VMEM((CH,), jnp.float32),
            pltpu.VMEM_SHARED((Npad,), jnp.float32),
            pltpu.VMEM((RZ,), jnp.float32),
        ]

    def body(q_hbm, w_hbm, src_hbm, dst_hbm, *rest):
        if with_deg:
            (ev_hbm, z_hbm, z1_hbm, agg_out, deg_out,
             src_v, dst_v, w_v, q_v, agg_sh, sem, ev_v, deg_sh, deg_v) = rest
        else:
            (z_hbm, agg_out,
             src_v, dst_v, w_v, q_v, agg_sh, sem) = rest

        c = lax.axis_index("c")
        s = lax.axis_index("s")
        wid = s * 2 + c
        zb = pl.multiple_of(s * RZ, 8)

        pltpu.sync_copy(z_hbm.at[pl.ds(zb, RZ)], agg_sh.at[pl.ds(zb, RZ)])
        if with_deg:
            pltpu.sync_copy(z1_hbm.at[pl.ds(zb, RZ)], deg_v)
            pltpu.sync_copy(deg_v, deg_sh.at[pl.ds(zb, RZ)])
        plsc.subcore_barrier()

        base0 = wid * EPT

        def chunk(g, carry):
            b = base0 + g * CH
            pltpu.sync_copy(src_hbm.at[pl.ds(b, CH)], src_v)
            pltpu.sync_copy(dst_hbm.at[pl.ds(b, CH)], dst_v)
            pltpu.sync_copy(w_hbm.at[pl.ds(b, CH)], w_v)
            if with_deg:
                pltpu.sync_copy(ev_hbm.at[pl.ds(b, CH)], ev_v)
            pltpu.async_copy(q_hbm.at[src_v], q_v, sem).wait()

            def row(r, carry2):
                for cc in range(D // 16):
                    sl = pl.ds(cc * 16, 16)
                    w_v[r, sl] = w_v[r, sl] * q_v[r, sl]
                return carry2
            lax.fori_loop(0, CH, row, 0)

            pltpu.sync_copy(w_v, agg_sh.at[dst_v], add=True)
            if with_deg:
                pltpu.sync_copy(ev_v, deg_sh.at[dst_v], add=True)
            return carry
        lax.fori_loop(0, CHUNKS, chunk, 0)

        plsc.subcore_barrier()
        pltpu.sync_copy(agg_sh.at[pl.ds(zb, RZ)], agg_out.at[c, pl.ds(zb, RZ)])
        if with_deg:
            db = pl.multiple_of(c * Npad + zb, 8)
            pltpu.sync_copy(deg_sh.at[pl.ds(zb, RZ)], deg_v)
            pltpu.sync_copy(deg_v, deg_out.at[pl.ds(db, RZ)])

    return pl.kernel(body, mesh=mesh, out_type=outs, scratch_types=scratch)


def kernel(input, edge_index, edge_attr, W1, b1, W2, b2):
    N, D = input.shape
    E, DE = edge_attr.shape
    H = W1.shape[1]
    Epad = ((E + NW * CH - 1) // (NW * CH)) * (NW * CH)

    Npad = ((N + 127) // 128) * 128

    ea_p = jnp.pad(edge_attr, ((0, Epad - E), (0, 0)))
    src = jnp.pad(edge_index[0], (0, Epad - E))
    dst = jnp.pad(edge_index[1], (0, Epad - E))
    z = jnp.zeros((Npad, D), jnp.float32)
    z1 = jnp.zeros((Npad,), jnp.float32)
    ev = (jnp.arange(Epad, dtype=jnp.int32) < E).astype(jnp.float32)

    w_pad = _make_fnet(E, Epad, DE, H, D)(
        ea_p, W1, b1.reshape(1, H), W2, b2.reshape(1, D))

    BN = 2000 if N % 2000 == 0 else N
    q0 = _make_softmax(N, D, BN)(input)

    agg1, deg = _make_sc_pass(Npad, D, Epad, True)(q0, w_pad, src, dst, ev, z, z1)
    agg1 = agg1[:, :N]
    deg = deg.reshape(2, Npad)[:, :N].reshape(2, N, 1)
    q1 = _make_update(N, D, BN, True)(input, agg1[0], agg1[1], deg[0], deg[1])

    (agg2,) = _make_sc_pass(Npad, D, Epad, False)(q1, w_pad, src, dst, z)
    agg2 = agg2[:, :N]
    out = _make_update(N, D, BN, False)(input, agg2[0], agg2[1], deg[0], deg[1])
    return out
